# baseline (speedup 1.0000x reference)
import functools

import jax
import jax.numpy as jnp
from jax import lax
from jax.experimental import pallas as pl
from jax.experimental.pallas import tpu as pltpu
from jax.experimental.pallas import tpu_sc as plsc

BATCH = 16384
HIST = 50
EMBED_DIM = 64
TOTAL = BATCH * HIST
NUM_CORES = 2
NUM_SUBCORES = 16
NUM_WORKERS = NUM_CORES * NUM_SUBCORES
PER_WORKER = TOTAL // NUM_WORKERS
CHUNK = 128
CHUNKS_PER_WORKER = PER_WORKER // CHUNK
SCALE = 8.0
LANES = 16

@functools.cache
def _build():
    mesh = plsc.VectorSubcoreMesh(core_axis_name="c", subcore_axis_name="s")

    @functools.partial(
        pl.kernel,
        mesh=mesh,
        out_type=jax.ShapeDtypeStruct((TOTAL, EMBED_DIM), jnp.float32),
        scratch_types=[
            pltpu.VMEM((CHUNKS_PER_WORKER, CHUNK), jnp.int32),
            pltpu.VMEM((CHUNK, EMBED_DIM), jnp.float32),
            pltpu.SemaphoreType.DMA,
        ],
        compiler_params=pltpu.CompilerParams(use_tc_tiling_on_sc=False),
    )
    def _gather_scale(x_hbm, tab_hbm, out_hbm, idx_v, rows_v, sem):
        wid = lax.axis_index("s") * NUM_CORES + lax.axis_index("c")
        base = wid * PER_WORKER
        pltpu.sync_copy(x_hbm.at[wid], idx_v)

        def chunk_body(g, carry):
            pltpu.async_copy(tab_hbm.at[idx_v.at[g]], rows_v, sem).wait()

            def scale_row(r, c2):
                for j in range(EMBED_DIM // LANES):
                    sl = pl.ds(j * LANES, LANES)
                    rows_v[r, sl] = rows_v[r, sl] * SCALE
                return c2

            lax.fori_loop(0, CHUNK, scale_row, 0, unroll=2)

            pltpu.sync_copy(rows_v, out_hbm.at[pl.ds(base + g * CHUNK, CHUNK)])
            return carry

        lax.fori_loop(0, CHUNKS_PER_WORKER, chunk_body, 0)

    return _gather_scale


def kernel(x, input_embedding):
    xf = x.reshape(NUM_WORKERS, CHUNKS_PER_WORKER, CHUNK)
    out = _build()(xf, input_embedding)
    return out.reshape(BATCH, HIST, EMBED_DIM)


# speedup vs baseline: 1.1609x; 1.1609x over previous
import functools

import jax
import jax.numpy as jnp
from jax import lax
from jax.experimental import pallas as pl
from jax.experimental.pallas import tpu as pltpu
from jax.experimental.pallas import tpu_sc as plsc

BATCH = 16384
HIST = 50
EMBED_DIM = 64
TOTAL = BATCH * HIST
NUM_CORES = 2
NUM_SUBCORES = 16
NUM_WORKERS = NUM_CORES * NUM_SUBCORES
PER_WORKER = TOTAL // NUM_WORKERS
CHUNK = 128
CHUNKS_PER_WORKER = PER_WORKER // CHUNK
SCALE = 8.0
LANES = 16

NBUF = 8
AHEAD = 6


@functools.cache
def _build():
    mesh = plsc.VectorSubcoreMesh(core_axis_name="c", subcore_axis_name="s")

    @functools.partial(
        pl.kernel,
        mesh=mesh,
        out_type=jax.ShapeDtypeStruct((TOTAL, EMBED_DIM), jnp.float32),
        scratch_types=[
            pltpu.VMEM((CHUNKS_PER_WORKER, CHUNK), jnp.int32),
            pltpu.VMEM((NBUF, CHUNK, EMBED_DIM), jnp.float32),
            pltpu.SemaphoreType.DMA((NBUF,)),
            pltpu.SemaphoreType.DMA((NBUF,)),
        ],
        compiler_params=pltpu.CompilerParams(use_tc_tiling_on_sc=False),
    )
    def _gather_scale(x_hbm, tab_hbm, out_hbm, idx_v, rows_v, gsem, ssem):
        wid = lax.axis_index("s") * NUM_CORES + lax.axis_index("c")
        base = wid * PER_WORKER
        pltpu.sync_copy(x_hbm.at[wid], idx_v)

        def gather(g, b):
            pltpu.async_copy(tab_hbm.at[idx_v.at[g]], rows_v.at[b], gsem.at[b])

        for b in range(AHEAD):
            gather(b, b)

        def outer(go):
            for b in range(NBUF):
                g = go + b
                pltpu.make_async_copy(
                    tab_hbm.at[idx_v.at[g]], rows_v.at[b], gsem.at[b]
                ).wait()

                def scale_row(r, c2, _b=b):
                    for j in range(EMBED_DIM // LANES):
                        sl = pl.ds(j * LANES, LANES)
                        rows_v[_b, r, sl] = rows_v[_b, r, sl] * SCALE
                    return c2

                lax.fori_loop(0, CHUNK, scale_row, 0, unroll=2)

                out_slice = out_hbm.at[pl.ds(base + g * CHUNK, CHUNK)]
                pltpu.async_copy(rows_v.at[b], out_slice, ssem.at[b])

                gn = g + AHEAD
                bq = (b + AHEAD) % NBUF

                @pl.when(gn < CHUNKS_PER_WORKER)
                def _():
                    @pl.when(gn >= NBUF)
                    def _():
                        pltpu.make_async_copy(
                            rows_v.at[bq],
                            out_hbm.at[pl.ds(base + (gn - NBUF) * CHUNK, CHUNK)],
                            ssem.at[bq],
                        ).wait()

                    gather(gn, bq)

        pl.loop(0, CHUNKS_PER_WORKER, step=NBUF)(outer)

        for b in range(NBUF):
            g = CHUNKS_PER_WORKER - NBUF + b
            pltpu.make_async_copy(
                rows_v.at[b],
                out_hbm.at[pl.ds(base + g * CHUNK, CHUNK)],
                ssem.at[b],
            ).wait()

    return _gather_scale


def kernel(x, input_embedding):
    xf = x.reshape(NUM_WORKERS, CHUNKS_PER_WORKER, CHUNK)
    out = _build()(xf, input_embedding)
    return out.reshape(BATCH, HIST, EMBED_DIM)
